# baseline (speedup 1.0000x reference)
import functools

import jax
import jax.numpy as jnp
from jax import lax
from jax.experimental import pallas as pl
from jax.experimental.pallas import tpu as pltpu
from jax.experimental.pallas import tpu_sc as plsc

B = 16384
N_FIELDS = 26
VOCAB = 100000
EMB_DIM = 32

NC = 2
NS = 16
LANES = 16
NW = NC * NS

TOTAL = B * N_FIELDS
PER_W = TOTAL // NW
CHUNK = 1664
NCHUNK = PER_W // CHUNK


def _gather(table_hbm, idx_ref, rows_ref, sem, chunk_id):
    return pltpu.make_async_copy(
        table_hbm.at[idx_ref.at[pl.ds(chunk_id * CHUNK, CHUNK)]],
        rows_ref,
        sem,
    )


@functools.partial(
    pl.kernel,
    out_type=jax.ShapeDtypeStruct((TOTAL, EMB_DIM), jnp.float32),
    mesh=plsc.VectorSubcoreMesh(
        core_axis_name="c", subcore_axis_name="s", num_cores=NC, num_subcores=NS
    ),
    scratch_types=[
        pltpu.VMEM((PER_W,), jnp.int32),
        pltpu.VMEM((CHUNK, EMB_DIM), jnp.float32),
        pltpu.VMEM((CHUNK, EMB_DIM), jnp.float32),
        pltpu.SemaphoreType.DMA,
        pltpu.SemaphoreType.DMA,
        pltpu.SemaphoreType.DMA,
        pltpu.SemaphoreType.DMA,
    ],
    compiler_params=pltpu.CompilerParams(use_tc_tiling_on_sc=False),
)
def _sc_gather(feat_hbm, table_hbm, out_hbm, idx_v, rows_a, rows_b, gsem_a,
               gsem_b, osem_a, osem_b):
    wid = lax.axis_index("s") * NC + lax.axis_index("c")
    wbase = wid * PER_W

    pltpu.sync_copy(feat_hbm.at[pl.ds(wbase, PER_W)], idx_v)
    lane = lax.iota(jnp.int32, LANES)

    def offsets_body(j, carry):
        pos = j * LANES + lane
        field = lax.rem(pos, N_FIELDS)
        cur = idx_v[pl.ds(j * LANES, LANES)]
        idx_v[pl.ds(j * LANES, LANES)] = cur + field * VOCAB
        return carry

    lax.fori_loop(0, PER_W // LANES, offsets_body, 0)

    def wb(rows_ref, sem, chunk_id):
        return pltpu.make_async_copy(
            rows_ref, out_hbm.at[pl.ds(wbase + chunk_id * CHUNK, CHUNK)], sem
        )

    _gather(table_hbm, idx_v, rows_a, gsem_a, 0).start()

    def step(t, carry):
        a = 2 * t
        @pl.when(t > 0)
        def _():
            wb(rows_b, osem_b, 0).wait()
        _gather(table_hbm, idx_v, rows_b, gsem_b, a + 1).start()
        _gather(table_hbm, idx_v, rows_a, gsem_a, 0).wait()
        wb(rows_a, osem_a, a).start()

        @pl.when(t < NCHUNK // 2 - 1)
        def _():
            wb(rows_a, osem_a, 0).wait()
            _gather(table_hbm, idx_v, rows_a, gsem_a, a + 2).start()
        _gather(table_hbm, idx_v, rows_b, gsem_b, 0).wait()
        wb(rows_b, osem_b, a + 1).start()
        return carry

    lax.fori_loop(0, NCHUNK // 2, step, 0)
    wb(rows_a, osem_a, 0).wait()
    wb(rows_b, osem_b, 0).wait()


def kernel(features, tables):
    feat_flat = features.reshape(TOTAL)
    table_flat = tables.reshape(N_FIELDS * VOCAB, EMB_DIM)
    out = _sc_gather(feat_flat, table_flat)
    return out.reshape(B, N_FIELDS * EMB_DIM)


# speedup vs baseline: 3.6242x; 3.6242x over previous
import functools

import jax
import jax.numpy as jnp
from jax import lax
from jax.experimental import pallas as pl
from jax.experimental.pallas import tpu as pltpu
from jax.experimental.pallas import tpu_sc as plsc

B = 16384
N_FIELDS = 26
VOCAB = 100000
EMB_DIM = 32

NC = 2
NS = 16
LANES = 16
NW = NC * NS

ROWS = N_FIELDS * EMB_DIM
PER_W = ROWS // NW
BCHUNK = 4096
NBCHUNK = B // BCHUNK
UNROLL = 8


@functools.partial(
    pl.kernel,
    out_type=jax.ShapeDtypeStruct((ROWS, B), jnp.float32),
    mesh=plsc.VectorSubcoreMesh(
        core_axis_name="c", subcore_axis_name="s", num_cores=NC, num_subcores=NS
    ),
    scratch_types=[
        pltpu.VMEM((VOCAB,), jnp.float32),
        pltpu.VMEM((BCHUNK,), jnp.int32),
        pltpu.VMEM((BCHUNK,), jnp.float32),
        pltpu.SemaphoreType.DMA,
    ],
    compiler_params=pltpu.CompilerParams(
        use_tc_tiling_on_sc=True, needs_layout_passes=False
    ),
)
def _sc_col_gather(feat_hbm, table_hbm, out_hbm, plane_v, fcol_v, out_v, sem):
    wid = lax.axis_index("s") * NC + lax.axis_index("c")

    def pair_body(k, carry):
        p = wid * PER_W + k
        f = p // EMB_DIM
        pltpu.async_copy(table_hbm.at[p], plane_v, sem).wait()

        def chunk_body(c, carry2):
            pltpu.async_copy(
                feat_hbm.at[f, pl.ds(c * BCHUNK, BCHUNK)], fcol_v, sem
            ).wait()

            def gather_body(g, carry3):
                for u in range(UNROLL):
                    off = (g * UNROLL + u) * LANES
                    idx = fcol_v[pl.ds(off, LANES)]
                    out_v[pl.ds(off, LANES)] = plsc.load_gather(
                        plane_v, [idx]
                    )
                return carry3

            lax.fori_loop(0, BCHUNK // (UNROLL * LANES), gather_body, 0)
            pltpu.async_copy(
                out_v, out_hbm.at[p, pl.ds(c * BCHUNK, BCHUNK)], sem
            ).wait()
            return carry2

        lax.fori_loop(0, NBCHUNK, chunk_body, 0)
        return carry

    lax.fori_loop(0, PER_W, pair_body, 0)


def kernel(features, tables):
    feat_t = features.T
    table_rows = tables.transpose(0, 2, 1).reshape(ROWS, VOCAB)
    out_t = _sc_col_gather(feat_t, table_rows)
    return out_t.T
